# Initial kernel scaffold; baseline (speedup 1.0000x reference)
import jax
import jax.numpy as jnp
from jax.experimental import pallas as pl


def kernel(z, W, emb):
    raise NotImplementedError("write your pallas kernel here")


# speedup vs baseline: 1.7825x; 1.7825x over previous
import jax
import jax.numpy as jnp
from jax import lax
from jax.experimental import pallas as pl
from jax.experimental.pallas import tpu as pltpu
from jax.experimental.pallas import tpu_sc as plsc

_B, _C_IN, _N = 4, 384, 576
_D, _K = 64, 512

_NC, _NS = 2, 16
_NW = _NC * _NS
_BN = _B * _N
_PW = _BN // _NW
_KR = _K // 16


def _tc_body(z_ref, w_ref, emb_ref, md_ref, mi_ref, zn_ref, en_ref):
    zb = z_ref[0]
    w = w_ref[...]
    emb = emb_ref[...]
    ze = lax.dot_general(w, zb, (((1,), (0,)), ((), ())))
    dot = lax.dot_general(emb, ze, (((1,), (0,)), ((), ())),
                          precision=lax.Precision.HIGHEST)
    emb2 = jnp.sum(emb * emb, axis=1, keepdims=True)
    ze2 = jnp.sum(ze * ze, axis=0, keepdims=True)
    scores = emb2 - 2.0 * dot
    smin = jnp.min(scores, axis=0, keepdims=True)
    kio = lax.broadcasted_iota(jnp.int32, (_K, _N), 0)
    mi = jnp.min(jnp.where(scores == smin, kio, _K), axis=0, keepdims=True)
    md_ref[...] = (smin + ze2)[None]
    mi_ref[...] = mi[None]
    zn_ref[...] = jnp.sqrt(ze2)[None]
    en_ref[...] = jnp.sqrt(emb2)


_tc_stage = pl.pallas_call(
    _tc_body,
    grid=(_B,),
    in_specs=[
        pl.BlockSpec((1, _C_IN, _N), lambda b: (b, 0, 0)),
        pl.BlockSpec((_D, _C_IN), lambda b: (0, 0)),
        pl.BlockSpec((_K, _D), lambda b: (0, 0)),
    ],
    out_specs=[
        pl.BlockSpec((1, 1, _N), lambda b: (b, 0, 0)),
        pl.BlockSpec((1, 1, _N), lambda b: (b, 0, 0)),
        pl.BlockSpec((1, 1, _N), lambda b: (b, 0, 0)),
        pl.BlockSpec((_K, 1), lambda b: (0, 0)),
    ],
    out_shape=[
        jax.ShapeDtypeStruct((_B, 1, _N), jnp.float32),
        jax.ShapeDtypeStruct((_B, 1, _N), jnp.int32),
        jax.ShapeDtypeStruct((_B, 1, _N), jnp.float32),
        jax.ShapeDtypeStruct((_K, 1), jnp.float32),
    ],
)


def _sc_body(emb_hbm, idx_hbm, rows_out, hist_out,
             idx_v, rows_v, hist_v, hist_m, id_v, shist, sem):
    cid = lax.axis_index("c")
    sid = lax.axis_index("s")
    wid = sid * _NC + cid
    base = wid * _PW

    lanes = lax.iota(jnp.int32, 16)
    zeros = jnp.zeros((16,), jnp.float32)
    ones = jnp.ones((16,), jnp.float32)

    for r in range(_KR):
        hist_v[pl.ds(r * 16, 16)] = zeros
        hist_m[r] = zeros
    id_v[pl.ds(0, 16)] = lanes
    id_v[pl.ds(16, 16)] = lanes + 16

    @pl.when(sid == 0)
    def _():
        pltpu.sync_copy(hist_m, shist)

    plsc.subcore_barrier()

    pltpu.sync_copy(idx_hbm.at[pl.ds(base, _PW)], idx_v)
    pltpu.async_copy(emb_hbm.at[idx_v], rows_v, sem).wait()
    pltpu.sync_copy(rows_v, rows_out.at[pl.ds(base, _PW)])

    for off in (0, 16, 32, 48, 56):
        idx16 = idx_v[pl.ds(off, 16)]
        lo = 8 if off == 56 else 0
        for j in range(lo, 16):
            plsc.addupdate_scatter(hist_v, [idx16], ones, mask=lanes == j)

    for r in range(_KR):
        hist_m[r] = hist_v[pl.ds(r * 16, 16)]
    pltpu.sync_copy(hist_m, shist.at[id_v], add=True)
    plsc.subcore_barrier()

    @pl.when(sid == 0)
    def _():
        pltpu.sync_copy(shist, hist_out.at[cid])


_sc_stage_cache = []


def _sc_stage():
    if not _sc_stage_cache:
        _sc_stage_cache.append(pl.kernel(
            _sc_body,
            out_type=[
                jax.ShapeDtypeStruct((_BN, _D), jnp.float32),
                jax.ShapeDtypeStruct((_NC, _KR, 16), jnp.float32),
            ],
            mesh=plsc.VectorSubcoreMesh(core_axis_name="c",
                                        subcore_axis_name="s",
                                        num_cores=_NC, num_subcores=_NS),
            compiler_params=pltpu.CompilerParams(needs_layout_passes=False,
                                                 use_tc_tiling_on_sc=False),
            scratch_types=[
                pltpu.VMEM((_PW,), jnp.int32),
                pltpu.VMEM((_PW, _D), jnp.float32),
                pltpu.VMEM((_K,), jnp.float32),
                pltpu.VMEM((_KR, 16), jnp.float32),
                pltpu.VMEM((_KR,), jnp.int32),
                pltpu.VMEM_SHARED((_KR, 16), jnp.float32),
                pltpu.SemaphoreType.DMA,
            ],
        ))
    return _sc_stage_cache[0]


def kernel(z, W, emb):
    md3, mi3, zn3, en2 = _tc_stage(z, W, emb)
    min_dist = md3.reshape(_B, _N)
    ze_norm = zn3.reshape(_B, _N)
    emb_norm = en2.reshape(_K)
    idx = mi3.reshape(_BN)
    rows, hist_parts = _sc_stage()(emb, idx)
    zq = rows.reshape(_B, _N, _D).transpose(0, 2, 1)
    ind_hist = hist_parts.reshape(_NC, _K).sum(axis=0)
    return zq, min_dist, ind_hist, ze_norm, emb_norm
